# baseline (speedup 1.0000x reference)
import functools

import jax
import jax.numpy as jnp
from jax import lax
from jax.experimental import pallas as pl
from jax.experimental.pallas import tpu as pltpu
from jax.experimental.pallas import tpu_sc as plsc

B = 4
N = 8192
M = 8192

QS = 2048
TCQ = N - QS


NW = 32
WPB = NW // B
QPW = QS // WPB
L = 16


def _sc_body(q_hbm, r_hbm, out_hbm, qx, qy, qz, rx, ry, rz, oi):
    c = lax.axis_index("c")
    s = lax.axis_index("s")
    wid = s * 2 + c
    b = wid // WPB
    qbase = TCQ + (wid % WPB) * QPW

    pltpu.sync_copy(q_hbm.at[pl.ds(b * 3 * N + 0 * N + qbase, QPW)], qx)
    pltpu.sync_copy(q_hbm.at[pl.ds(b * 3 * N + 1 * N + qbase, QPW)], qy)
    pltpu.sync_copy(q_hbm.at[pl.ds(b * 3 * N + 2 * N + qbase, QPW)], qz)
    pltpu.sync_copy(r_hbm.at[pl.ds(b * 3 * M + 0 * M, M)], rx)
    pltpu.sync_copy(r_hbm.at[pl.ds(b * 3 * M + 1 * M, M)], ry)
    pltpu.sync_copy(r_hbm.at[pl.ds(b * 3 * M + 2 * M, M)], rz)

    NA = 1
    GV = 8

    def per_group(g, carry):
        qv = [
            (
                qx[pl.ds((g * GV + k) * L, L)],
                qy[pl.ds((g * GV + k) * L, L)],
                qz[pl.ds((g * GV + k) * L, L)],
            )
            for k in range(GV)
        ]

        def per_refvec(j, mc):
            acc = [list(a) for a in zip(mc[0::2], mc[1::2])]
            rxv = rx[pl.ds(j * L, L)]
            ryv = ry[pl.ds(j * L, L)]
            rzv = rz[pl.ds(j * L, L)]
            base = jnp.full((L,), j * L, jnp.int32)
            for t in range(L):
                tv = jnp.full((L,), t, jnp.int32)
                bx = jnp.take(rxv, tv)
                by = jnp.take(ryv, tv)
                bz = jnp.take(rzv, tv)
                for k in range(GV):
                    dx = qv[k][0] - bx
                    dy = qv[k][1] - by
                    dz = qv[k][2] - bz
                    d = dx * dx + dy * dy + dz * dz
                    a = k * NA + t % NA
                    m, mi = acc[a]
                    p = d < m
                    acc[a][0] = jnp.minimum(m, d)
                    acc[a][1] = jnp.where(p, base + t, mi)
            return tuple(x for a in acc for x in a)

        init = []
        for _ in range(GV * NA):
            init.append(jnp.full((L,), jnp.inf, jnp.float32))
            init.append(jnp.zeros((L,), jnp.int32))
        accs = lax.fori_loop(0, M // L, per_refvec, tuple(init))

        for k in range(GV):
            m, mi = accs[2 * k * NA], accs[2 * k * NA + 1]
            for a in range(1, NA):
                mb = accs[2 * (k * NA + a)]
                ib = accs[2 * (k * NA + a) + 1]
                takeb = (mb < m) | ((mb == m) & (ib < mi))
                m = jnp.minimum(m, mb)
                mi = jnp.where(takeb, ib, mi)
            oi[pl.ds((g * GV + k) * L, L)] = mi
        return carry

    lax.fori_loop(0, QPW // (GV * L), per_group, 0)
    pltpu.sync_copy(oi, out_hbm.at[pl.ds(wid * QPW, QPW)])


_sc_nn = functools.partial(
    pl.kernel,
    out_type=jax.ShapeDtypeStruct((B * QS,), jnp.int32),
    mesh=plsc.VectorSubcoreMesh(core_axis_name="c", subcore_axis_name="s"),
    scratch_types=[
        pltpu.VMEM((QPW,), jnp.float32),
        pltpu.VMEM((QPW,), jnp.float32),
        pltpu.VMEM((QPW,), jnp.float32),
        pltpu.VMEM((M,), jnp.float32),
        pltpu.VMEM((M,), jnp.float32),
        pltpu.VMEM((M,), jnp.float32),
        pltpu.VMEM((QPW,), jnp.int32),
    ],
)(_sc_body)


CH = 1024
KT = 6
QG = KT * 1024


def _tc_body(qx_ref, qy_ref, qz_ref, rxyz_ref, out_ref, m_ref, mi_ref):
    r = pl.program_id(2)

    @pl.when(r == 0)
    def _():
        m_ref[...] = jnp.full((KT * 8, 128), jnp.inf, jnp.float32)
        mi_ref[...] = jnp.zeros((KT * 8, 128), jnp.int32)

    qxt = qx_ref[0, 0]
    qyt = qy_ref[0, 0]
    qzt = qz_ref[0, 0]
    base = r * CH

    def step(j, mc):
        m, mi = mc
        rx = rxyz_ref[0, 0, 3 * j]
        ry = rxyz_ref[0, 0, 3 * j + 1]
        rz = rxyz_ref[0, 0, 3 * j + 2]
        dx = qxt - rx
        dy = qyt - ry
        dz = qzt - rz
        d = dx * dx + dy * dy + dz * dz
        p = d < m
        m = jnp.minimum(m, d)
        mi = jnp.where(p, jnp.full((KT * 8, 128), base + j, jnp.int32), mi)
        return m, mi

    m, mi = lax.fori_loop(0, CH, step, (m_ref[...], mi_ref[...]), unroll=4)
    m_ref[...] = m
    mi_ref[...] = mi

    @pl.when(r == M // CH - 1)
    def _():
        out_ref[0] = mi


def _tc_nn(q, r, nb):
    qt = q.reshape(nb, 3, N // 128, 128)
    nrc = M // CH
    rt = jnp.swapaxes(r, 1, 2).reshape(nb * nrc, 1, CH * 3)
    grid = (nb, TCQ // QG, nrc)

    out = pl.pallas_call(
        _tc_body,
        grid=grid,
        in_specs=[
            pl.BlockSpec((1, 1, QG // 128, 128), lambda b, i, j: (b, 0, i, 0)),
            pl.BlockSpec((1, 1, QG // 128, 128), lambda b, i, j: (b, 1, i, 0)),
            pl.BlockSpec((1, 1, QG // 128, 128), lambda b, i, j: (b, 2, i, 0)),
            pl.BlockSpec((1, 1, CH * 3), lambda b, i, j: (b * nrc + j, 0, 0),
                         memory_space=pltpu.SMEM),
        ],
        out_specs=pl.BlockSpec((1, QG // 128, 128), lambda b, i, j: (b, i, 0)),
        out_shape=jax.ShapeDtypeStruct((nb, TCQ // 128, 128), jnp.int32),
        scratch_shapes=[
            pltpu.VMEM((KT * 8, 128), jnp.float32),
            pltpu.VMEM((KT * 8, 128), jnp.int32),
        ],
        compiler_params=pltpu.CompilerParams(
            dimension_semantics=("arbitrary", "arbitrary", "arbitrary"),
        ),
    )(qt, qt, qt, rt)
    return out.reshape(nb, TCQ)


def kernel(S1, S2):
    q = S1.transpose(0, 2, 1)
    r = S2.transpose(0, 2, 1)
    idx_sc = _sc_nn(q.reshape(-1), r.reshape(-1)).reshape(B, QS)
    idx_tc = _tc_nn(q, r, B)
    idx = jnp.concatenate([idx_tc, idx_sc], axis=1)
    return idx.astype(jnp.int64)


# speedup vs baseline: 1.0744x; 1.0744x over previous
import functools

import jax
import jax.numpy as jnp
from jax import lax
from jax.experimental import pallas as pl
from jax.experimental.pallas import tpu as pltpu
from jax.experimental.pallas import tpu_sc as plsc

B = 4
N = 8192
M = 8192

QS = 2048
TCQ = N - QS


NW = 32
WPB = NW // B
QPW = QS // WPB
L = 16


def _sc_body(q_hbm, r_hbm, out_hbm, qx, qy, qz, rx, ry, rz, oi):
    c = lax.axis_index("c")
    s = lax.axis_index("s")
    wid = s * 2 + c
    b = wid // WPB
    qbase = TCQ + (wid % WPB) * QPW

    pltpu.sync_copy(q_hbm.at[pl.ds(b * 3 * N + 0 * N + qbase, QPW)], qx)
    pltpu.sync_copy(q_hbm.at[pl.ds(b * 3 * N + 1 * N + qbase, QPW)], qy)
    pltpu.sync_copy(q_hbm.at[pl.ds(b * 3 * N + 2 * N + qbase, QPW)], qz)
    pltpu.sync_copy(r_hbm.at[pl.ds(b * 3 * M + 0 * M, M)], rx)
    pltpu.sync_copy(r_hbm.at[pl.ds(b * 3 * M + 1 * M, M)], ry)
    pltpu.sync_copy(r_hbm.at[pl.ds(b * 3 * M + 2 * M, M)], rz)

    NA = 1
    GV = 8

    def per_group(g, carry):
        qv = [
            (
                qx[pl.ds((g * GV + k) * L, L)],
                qy[pl.ds((g * GV + k) * L, L)],
                qz[pl.ds((g * GV + k) * L, L)],
            )
            for k in range(GV)
        ]

        def per_refvec(j, mc):
            acc = [list(a) for a in zip(mc[0::2], mc[1::2])]
            rxv = rx[pl.ds(j * L, L)]
            ryv = ry[pl.ds(j * L, L)]
            rzv = rz[pl.ds(j * L, L)]
            base = jnp.full((L,), j * L, jnp.int32)
            for t in range(L):
                tv = jnp.full((L,), t, jnp.int32)
                bx = jnp.take(rxv, tv)
                by = jnp.take(ryv, tv)
                bz = jnp.take(rzv, tv)
                for k in range(GV):
                    dx = qv[k][0] - bx
                    dy = qv[k][1] - by
                    dz = qv[k][2] - bz
                    d = dx * dx + dy * dy + dz * dz
                    a = k * NA + t % NA
                    m, mi = acc[a]
                    p = d < m
                    acc[a][0] = jnp.minimum(m, d)
                    acc[a][1] = jnp.where(p, base + t, mi)
            return tuple(x for a in acc for x in a)

        init = []
        for _ in range(GV * NA):
            init.append(jnp.full((L,), jnp.inf, jnp.float32))
            init.append(jnp.zeros((L,), jnp.int32))
        accs = lax.fori_loop(0, M // L, per_refvec, tuple(init))

        for k in range(GV):
            m, mi = accs[2 * k * NA], accs[2 * k * NA + 1]
            for a in range(1, NA):
                mb = accs[2 * (k * NA + a)]
                ib = accs[2 * (k * NA + a) + 1]
                takeb = (mb < m) | ((mb == m) & (ib < mi))
                m = jnp.minimum(m, mb)
                mi = jnp.where(takeb, ib, mi)
            oi[pl.ds((g * GV + k) * L, L)] = mi
        return carry

    lax.fori_loop(0, QPW // (GV * L), per_group, 0)
    pltpu.sync_copy(oi, out_hbm.at[pl.ds(wid * QPW, QPW)])


_sc_nn = functools.partial(
    pl.kernel,
    out_type=jax.ShapeDtypeStruct((B * QS,), jnp.int32),
    mesh=plsc.VectorSubcoreMesh(core_axis_name="c", subcore_axis_name="s"),
    scratch_types=[
        pltpu.VMEM((QPW,), jnp.float32),
        pltpu.VMEM((QPW,), jnp.float32),
        pltpu.VMEM((QPW,), jnp.float32),
        pltpu.VMEM((M,), jnp.float32),
        pltpu.VMEM((M,), jnp.float32),
        pltpu.VMEM((M,), jnp.float32),
        pltpu.VMEM((QPW,), jnp.int32),
    ],
)(_sc_body)


CH = 1024
KT = 6
QG = KT * 1024


def _tc_body(qx_ref, qy_ref, qz_ref, rx_ref, ry_ref, rz_ref, out_ref, m_ref, mi_ref):
    r = pl.program_id(2)

    @pl.when(r == 0)
    def _():
        m_ref[...] = jnp.full((KT * 8, 128), jnp.inf, jnp.float32)
        mi_ref[...] = jnp.zeros((KT * 8, 128), jnp.int32)

    base = r * CH

    def step(j, mc):
        rx = rx_ref[0, 0, j]
        ry = ry_ref[0, 0, j]
        rz = rz_ref[0, 0, j]
        iv = jnp.full((8, 128), base + j, jnp.int32)
        out = []
        for k in range(KT):
            m, mi = mc[2 * k], mc[2 * k + 1]
            dx = qx_ref[0, 0, pl.ds(k * 8, 8), :] - rx
            dy = qy_ref[0, 0, pl.ds(k * 8, 8), :] - ry
            dz = qz_ref[0, 0, pl.ds(k * 8, 8), :] - rz
            d = dx * dx + dy * dy + dz * dz
            p = d < m
            out.append(jnp.minimum(m, d))
            out.append(jnp.where(p, iv, mi))
        return tuple(out)

    init = []
    for k in range(KT):
        init.append(m_ref[pl.ds(k * 8, 8), :])
        init.append(mi_ref[pl.ds(k * 8, 8), :])
    fin = lax.fori_loop(0, CH, step, tuple(init), unroll=4)
    for k in range(KT):
        m_ref[pl.ds(k * 8, 8), :] = fin[2 * k]
        mi_ref[pl.ds(k * 8, 8), :] = fin[2 * k + 1]

    @pl.when(r == M // CH - 1)
    def _():
        out_ref[0] = mi_ref[...]


def _tc_nn(q, r, nb):
    qt = q.reshape(nb, 3, N // 128, 128)
    nrc = M // CH
    rt = r.reshape(nb * 3 * nrc, 1, CH)
    grid = (nb, TCQ // QG, nrc)

    def rmap(c):
        return lambda b, i, j: ((b * 3 + c) * nrc + j, 0, 0)

    out = pl.pallas_call(
        _tc_body,
        grid=grid,
        in_specs=[
            pl.BlockSpec((1, 1, QG // 128, 128), lambda b, i, j: (b, 0, i, 0)),
            pl.BlockSpec((1, 1, QG // 128, 128), lambda b, i, j: (b, 1, i, 0)),
            pl.BlockSpec((1, 1, QG // 128, 128), lambda b, i, j: (b, 2, i, 0)),
            pl.BlockSpec((1, 1, CH), rmap(0), memory_space=pltpu.SMEM),
            pl.BlockSpec((1, 1, CH), rmap(1), memory_space=pltpu.SMEM),
            pl.BlockSpec((1, 1, CH), rmap(2), memory_space=pltpu.SMEM),
        ],
        out_specs=pl.BlockSpec((1, QG // 128, 128), lambda b, i, j: (b, i, 0)),
        out_shape=jax.ShapeDtypeStruct((nb, TCQ // 128, 128), jnp.int32),
        scratch_shapes=[
            pltpu.VMEM((KT * 8, 128), jnp.float32),
            pltpu.VMEM((KT * 8, 128), jnp.int32),
        ],
        compiler_params=pltpu.CompilerParams(
            dimension_semantics=("arbitrary", "arbitrary", "arbitrary"),
        ),
    )(qt, qt, qt, rt, rt, rt)
    return out.reshape(nb, TCQ)


def kernel(S1, S2):
    q = S1.transpose(0, 2, 1)
    r = S2.transpose(0, 2, 1)
    idx_sc = _sc_nn(q.reshape(-1), r.reshape(-1)).reshape(B, QS)
    idx_tc = _tc_nn(q, r, B)
    idx = jnp.concatenate([idx_tc, idx_sc], axis=1)
    return idx.astype(jnp.int64)


# speedup vs baseline: 1.1511x; 1.0714x over previous
import functools

import jax
import jax.numpy as jnp
from jax import lax
from jax.experimental import pallas as pl
from jax.experimental.pallas import tpu as pltpu
from jax.experimental.pallas import tpu_sc as plsc

B = 4
N = 8192
M = 8192

QS = 2048
TCQ = N - QS


NW = 32
WPB = NW // B
QPW = QS // WPB
L = 16


def _sc_body(q_hbm, r_hbm, out_hbm, qx, qy, qz, rx, ry, rz, oi):
    c = lax.axis_index("c")
    s = lax.axis_index("s")
    wid = s * 2 + c
    b = wid // WPB
    qbase = TCQ + (wid % WPB) * QPW

    pltpu.sync_copy(q_hbm.at[pl.ds(b * 3 * N + 0 * N + qbase, QPW)], qx)
    pltpu.sync_copy(q_hbm.at[pl.ds(b * 3 * N + 1 * N + qbase, QPW)], qy)
    pltpu.sync_copy(q_hbm.at[pl.ds(b * 3 * N + 2 * N + qbase, QPW)], qz)
    pltpu.sync_copy(r_hbm.at[pl.ds(b * 3 * M + 0 * M, M)], rx)
    pltpu.sync_copy(r_hbm.at[pl.ds(b * 3 * M + 1 * M, M)], ry)
    pltpu.sync_copy(r_hbm.at[pl.ds(b * 3 * M + 2 * M, M)], rz)

    NA = 1
    GV = 8

    def per_group(g, carry):
        qv = [
            (
                qx[pl.ds((g * GV + k) * L, L)],
                qy[pl.ds((g * GV + k) * L, L)],
                qz[pl.ds((g * GV + k) * L, L)],
            )
            for k in range(GV)
        ]

        def per_refvec(j, mc):
            acc = [list(a) for a in zip(mc[0::2], mc[1::2])]
            rxv = rx[pl.ds(j * L, L)]
            ryv = ry[pl.ds(j * L, L)]
            rzv = rz[pl.ds(j * L, L)]
            base = jnp.full((L,), j * L, jnp.int32)
            for t in range(L):
                tv = jnp.full((L,), t, jnp.int32)
                bx = jnp.take(rxv, tv)
                by = jnp.take(ryv, tv)
                bz = jnp.take(rzv, tv)
                for k in range(GV):
                    dx = qv[k][0] - bx
                    dy = qv[k][1] - by
                    dz = qv[k][2] - bz
                    d = dx * dx + dy * dy + dz * dz
                    a = k * NA + t % NA
                    m, mi = acc[a]
                    p = d < m
                    acc[a][0] = jnp.minimum(m, d)
                    acc[a][1] = jnp.where(p, base + t, mi)
            return tuple(x for a in acc for x in a)

        init = []
        for _ in range(GV * NA):
            init.append(jnp.full((L,), jnp.inf, jnp.float32))
            init.append(jnp.zeros((L,), jnp.int32))
        accs = lax.fori_loop(0, M // L, per_refvec, tuple(init))

        for k in range(GV):
            m, mi = accs[2 * k * NA], accs[2 * k * NA + 1]
            for a in range(1, NA):
                mb = accs[2 * (k * NA + a)]
                ib = accs[2 * (k * NA + a) + 1]
                takeb = (mb < m) | ((mb == m) & (ib < mi))
                m = jnp.minimum(m, mb)
                mi = jnp.where(takeb, ib, mi)
            oi[pl.ds((g * GV + k) * L, L)] = mi
        return carry

    lax.fori_loop(0, QPW // (GV * L), per_group, 0)
    pltpu.sync_copy(oi, out_hbm.at[pl.ds(wid * QPW, QPW)])


_sc_nn = functools.partial(
    pl.kernel,
    out_type=jax.ShapeDtypeStruct((B * QS,), jnp.int32),
    mesh=plsc.VectorSubcoreMesh(core_axis_name="c", subcore_axis_name="s"),
    scratch_types=[
        pltpu.VMEM((QPW,), jnp.float32),
        pltpu.VMEM((QPW,), jnp.float32),
        pltpu.VMEM((QPW,), jnp.float32),
        pltpu.VMEM((M,), jnp.float32),
        pltpu.VMEM((M,), jnp.float32),
        pltpu.VMEM((M,), jnp.float32),
        pltpu.VMEM((QPW,), jnp.int32),
    ],
)(_sc_body)


CH = 1024
KT = 6
QG = KT * 1024


def _tc_body(qx_ref, qy_ref, qz_ref, rx_ref, ry_ref, rz_ref, out_ref, m_ref, mi_ref):
    r = pl.program_id(2)

    @pl.when(r == 0)
    def _():
        m_ref[...] = jnp.full((KT * 8, 128), jnp.inf, jnp.float32)
        mi_ref[...] = jnp.zeros((KT * 8, 128), jnp.int32)

    qxt = qx_ref[0, 0]
    qyt = qy_ref[0, 0]
    qzt = qz_ref[0, 0]
    base = r * CH

    def step(j, mc):
        m, mi = mc
        rx = rx_ref[0, 0, j]
        ry = ry_ref[0, 0, j]
        rz = rz_ref[0, 0, j]
        dx = qxt - rx
        dy = qyt - ry
        dz = qzt - rz
        d = dx * dx + dy * dy + dz * dz
        p = d < m
        m = jnp.minimum(m, d)
        mi = jnp.where(p, jnp.full((KT * 8, 128), base + j, jnp.int32), mi)
        return m, mi

    m, mi = lax.fori_loop(0, CH, step, (m_ref[...], mi_ref[...]), unroll=8)
    m_ref[...] = m
    mi_ref[...] = mi

    @pl.when(r == M // CH - 1)
    def _():
        out_ref[0] = mi


def _tc_nn(q, r, nb):
    qt = q.reshape(nb, 3, N // 128, 128)
    nrc = M // CH
    rt = r.reshape(nb * 3 * nrc, 1, CH)
    grid = (nb, TCQ // QG, nrc)

    def rmap(c):
        return lambda b, i, j: ((b * 3 + c) * nrc + j, 0, 0)

    out = pl.pallas_call(
        _tc_body,
        grid=grid,
        in_specs=[
            pl.BlockSpec((1, 1, QG // 128, 128), lambda b, i, j: (b, 0, i, 0)),
            pl.BlockSpec((1, 1, QG // 128, 128), lambda b, i, j: (b, 1, i, 0)),
            pl.BlockSpec((1, 1, QG // 128, 128), lambda b, i, j: (b, 2, i, 0)),
            pl.BlockSpec((1, 1, CH), rmap(0), memory_space=pltpu.SMEM),
            pl.BlockSpec((1, 1, CH), rmap(1), memory_space=pltpu.SMEM),
            pl.BlockSpec((1, 1, CH), rmap(2), memory_space=pltpu.SMEM),
        ],
        out_specs=pl.BlockSpec((1, QG // 128, 128), lambda b, i, j: (b, i, 0)),
        out_shape=jax.ShapeDtypeStruct((nb, TCQ // 128, 128), jnp.int32),
        scratch_shapes=[
            pltpu.VMEM((KT * 8, 128), jnp.float32),
            pltpu.VMEM((KT * 8, 128), jnp.int32),
        ],
        compiler_params=pltpu.CompilerParams(
            dimension_semantics=("arbitrary", "arbitrary", "arbitrary"),
        ),
    )(qt, qt, qt, rt, rt, rt)
    return out.reshape(nb, TCQ)


def kernel(S1, S2):
    q = S1.transpose(0, 2, 1)
    r = S2.transpose(0, 2, 1)
    idx_sc = _sc_nn(q.reshape(-1), r.reshape(-1)).reshape(B, QS)
    idx_tc = _tc_nn(q, r, B)
    idx = jnp.concatenate([idx_tc, idx_sc], axis=1)
    return idx.astype(jnp.int64)


# speedup vs baseline: 1.1742x; 1.0201x over previous
import functools

import jax
import jax.numpy as jnp
from jax import lax
from jax.experimental import pallas as pl
from jax.experimental.pallas import tpu as pltpu
from jax.experimental.pallas import tpu_sc as plsc

B = 4
N = 8192
M = 8192

QS = 2048
TCQ = N - QS


NW = 32
WPB = NW // B
QPW = QS // WPB
L = 16


def _sc_body(q_hbm, r_hbm, out_hbm, qx, qy, qz, rx, ry, rz, oi):
    c = lax.axis_index("c")
    s = lax.axis_index("s")
    wid = s * 2 + c
    b = wid // WPB
    qbase = TCQ + (wid % WPB) * QPW

    pltpu.sync_copy(q_hbm.at[pl.ds(b * 3 * N + 0 * N + qbase, QPW)], qx)
    pltpu.sync_copy(q_hbm.at[pl.ds(b * 3 * N + 1 * N + qbase, QPW)], qy)
    pltpu.sync_copy(q_hbm.at[pl.ds(b * 3 * N + 2 * N + qbase, QPW)], qz)
    pltpu.sync_copy(r_hbm.at[pl.ds(b * 3 * M + 0 * M, M)], rx)
    pltpu.sync_copy(r_hbm.at[pl.ds(b * 3 * M + 1 * M, M)], ry)
    pltpu.sync_copy(r_hbm.at[pl.ds(b * 3 * M + 2 * M, M)], rz)

    NA = 1
    GV = 8

    def per_group(g, carry):
        qv = [
            (
                qx[pl.ds((g * GV + k) * L, L)],
                qy[pl.ds((g * GV + k) * L, L)],
                qz[pl.ds((g * GV + k) * L, L)],
            )
            for k in range(GV)
        ]

        def per_refvec(j, mc):
            acc = [list(a) for a in zip(mc[0::2], mc[1::2])]
            rxv = rx[pl.ds(j * L, L)]
            ryv = ry[pl.ds(j * L, L)]
            rzv = rz[pl.ds(j * L, L)]
            base = jnp.full((L,), j * L, jnp.int32)
            for t in range(L):
                tv = jnp.full((L,), t, jnp.int32)
                bx = jnp.take(rxv, tv)
                by = jnp.take(ryv, tv)
                bz = jnp.take(rzv, tv)
                for k in range(GV):
                    dx = qv[k][0] - bx
                    dy = qv[k][1] - by
                    dz = qv[k][2] - bz
                    d = dx * dx + dy * dy + dz * dz
                    a = k * NA + t % NA
                    m, mi = acc[a]
                    p = d < m
                    acc[a][0] = jnp.minimum(m, d)
                    acc[a][1] = jnp.where(p, base + t, mi)
            return tuple(x for a in acc for x in a)

        init = []
        for _ in range(GV * NA):
            init.append(jnp.full((L,), jnp.inf, jnp.float32))
            init.append(jnp.zeros((L,), jnp.int32))
        accs = lax.fori_loop(0, M // L, per_refvec, tuple(init))

        for k in range(GV):
            m, mi = accs[2 * k * NA], accs[2 * k * NA + 1]
            for a in range(1, NA):
                mb = accs[2 * (k * NA + a)]
                ib = accs[2 * (k * NA + a) + 1]
                takeb = (mb < m) | ((mb == m) & (ib < mi))
                m = jnp.minimum(m, mb)
                mi = jnp.where(takeb, ib, mi)
            oi[pl.ds((g * GV + k) * L, L)] = mi
        return carry

    lax.fori_loop(0, QPW // (GV * L), per_group, 0)
    pltpu.sync_copy(oi, out_hbm.at[pl.ds(wid * QPW, QPW)])


_sc_nn = functools.partial(
    pl.kernel,
    out_type=jax.ShapeDtypeStruct((B * QS,), jnp.int32),
    mesh=plsc.VectorSubcoreMesh(core_axis_name="c", subcore_axis_name="s"),
    scratch_types=[
        pltpu.VMEM((QPW,), jnp.float32),
        pltpu.VMEM((QPW,), jnp.float32),
        pltpu.VMEM((QPW,), jnp.float32),
        pltpu.VMEM((M,), jnp.float32),
        pltpu.VMEM((M,), jnp.float32),
        pltpu.VMEM((M,), jnp.float32),
        pltpu.VMEM((QPW,), jnp.int32),
    ],
)(_sc_body)


CH = 1024
KT = 6
QG = KT * 1024


def _tc_body(qx_ref, qy_ref, qz_ref, rx_ref, ry_ref, rz_ref, out_ref, m_ref, mi_ref):
    r = pl.program_id(2)

    @pl.when(r == 0)
    def _():
        m_ref[...] = jnp.full((KT * 8, 128), jnp.inf, jnp.float32)
        mi_ref[...] = jnp.zeros((KT * 8, 128), jnp.int32)

    qxt = qx_ref[0, 0]
    qyt = qy_ref[0, 0]
    qzt = qz_ref[0, 0]
    base = r * CH

    def step(j, mc):
        m, mi = mc
        rx = rx_ref[0, 0, j]
        ry = ry_ref[0, 0, j]
        rz = rz_ref[0, 0, j]
        dx = qxt - rx
        dy = qyt - ry
        dz = qzt - rz
        d = dx * dx + dy * dy + dz * dz
        p = d < m
        m = jnp.minimum(m, d)
        mi = jnp.where(p, jnp.full((KT * 8, 128), base + j, jnp.int32), mi)
        return m, mi

    m, mi = lax.fori_loop(0, CH, step, (m_ref[...], mi_ref[...]), unroll=16)
    m_ref[...] = m
    mi_ref[...] = mi

    @pl.when(r == M // CH - 1)
    def _():
        out_ref[0] = mi


def _tc_nn(q, r, nb):
    qt = q.reshape(nb, 3, N // 128, 128)
    nrc = M // CH
    rt = r.reshape(nb * 3 * nrc, 1, CH)
    grid = (nb, TCQ // QG, nrc)

    def rmap(c):
        return lambda b, i, j: ((b * 3 + c) * nrc + j, 0, 0)

    out = pl.pallas_call(
        _tc_body,
        grid=grid,
        in_specs=[
            pl.BlockSpec((1, 1, QG // 128, 128), lambda b, i, j: (b, 0, i, 0)),
            pl.BlockSpec((1, 1, QG // 128, 128), lambda b, i, j: (b, 1, i, 0)),
            pl.BlockSpec((1, 1, QG // 128, 128), lambda b, i, j: (b, 2, i, 0)),
            pl.BlockSpec((1, 1, CH), rmap(0), memory_space=pltpu.SMEM),
            pl.BlockSpec((1, 1, CH), rmap(1), memory_space=pltpu.SMEM),
            pl.BlockSpec((1, 1, CH), rmap(2), memory_space=pltpu.SMEM),
        ],
        out_specs=pl.BlockSpec((1, QG // 128, 128), lambda b, i, j: (b, i, 0)),
        out_shape=jax.ShapeDtypeStruct((nb, TCQ // 128, 128), jnp.int32),
        scratch_shapes=[
            pltpu.VMEM((KT * 8, 128), jnp.float32),
            pltpu.VMEM((KT * 8, 128), jnp.int32),
        ],
        compiler_params=pltpu.CompilerParams(
            dimension_semantics=("arbitrary", "arbitrary", "arbitrary"),
        ),
    )(qt, qt, qt, rt, rt, rt)
    return out.reshape(nb, TCQ)


def kernel(S1, S2):
    q = S1.transpose(0, 2, 1)
    r = S2.transpose(0, 2, 1)
    idx_sc = _sc_nn(q.reshape(-1), r.reshape(-1)).reshape(B, QS)
    idx_tc = _tc_nn(q, r, B)
    idx = jnp.concatenate([idx_tc, idx_sc], axis=1)
    return idx.astype(jnp.int64)


# speedup vs baseline: 1.2004x; 1.0223x over previous
import functools

import jax
import jax.numpy as jnp
from jax import lax
from jax.experimental import pallas as pl
from jax.experimental.pallas import tpu as pltpu
from jax.experimental.pallas import tpu_sc as plsc

B = 4
N = 8192
M = 8192

QS = 2048
TCQ = N - QS


NW = 32
WPB = NW // B
QPW = QS // WPB
L = 16


def _sc_body(q_hbm, r_hbm, out_hbm, qx, qy, qz, rx, ry, rz, oi):
    c = lax.axis_index("c")
    s = lax.axis_index("s")
    wid = s * 2 + c
    b = wid // WPB
    qbase = TCQ + (wid % WPB) * QPW

    pltpu.sync_copy(q_hbm.at[pl.ds(b * 3 * N + 0 * N + qbase, QPW)], qx)
    pltpu.sync_copy(q_hbm.at[pl.ds(b * 3 * N + 1 * N + qbase, QPW)], qy)
    pltpu.sync_copy(q_hbm.at[pl.ds(b * 3 * N + 2 * N + qbase, QPW)], qz)
    pltpu.sync_copy(r_hbm.at[pl.ds(b * 3 * M + 0 * M, M)], rx)
    pltpu.sync_copy(r_hbm.at[pl.ds(b * 3 * M + 1 * M, M)], ry)
    pltpu.sync_copy(r_hbm.at[pl.ds(b * 3 * M + 2 * M, M)], rz)

    NA = 1
    GV = 8

    def per_group(g, carry):
        qv = [
            (
                qx[pl.ds((g * GV + k) * L, L)],
                qy[pl.ds((g * GV + k) * L, L)],
                qz[pl.ds((g * GV + k) * L, L)],
            )
            for k in range(GV)
        ]

        def per_refvec(j, mc):
            acc = [list(a) for a in zip(mc[0::2], mc[1::2])]
            rxv = rx[pl.ds(j * L, L)]
            ryv = ry[pl.ds(j * L, L)]
            rzv = rz[pl.ds(j * L, L)]
            base = jnp.full((L,), j * L, jnp.int32)
            for t in range(L):
                tv = jnp.full((L,), t, jnp.int32)
                bx = jnp.take(rxv, tv)
                by = jnp.take(ryv, tv)
                bz = jnp.take(rzv, tv)
                for k in range(GV):
                    dx = qv[k][0] - bx
                    dy = qv[k][1] - by
                    dz = qv[k][2] - bz
                    d = dx * dx + dy * dy + dz * dz
                    a = k * NA + t % NA
                    m, mi = acc[a]
                    p = d < m
                    acc[a][0] = jnp.minimum(m, d)
                    acc[a][1] = jnp.where(p, base + t, mi)
            return tuple(x for a in acc for x in a)

        init = []
        for _ in range(GV * NA):
            init.append(jnp.full((L,), jnp.inf, jnp.float32))
            init.append(jnp.zeros((L,), jnp.int32))
        accs = lax.fori_loop(0, M // L, per_refvec, tuple(init))

        for k in range(GV):
            m, mi = accs[2 * k * NA], accs[2 * k * NA + 1]
            for a in range(1, NA):
                mb = accs[2 * (k * NA + a)]
                ib = accs[2 * (k * NA + a) + 1]
                takeb = (mb < m) | ((mb == m) & (ib < mi))
                m = jnp.minimum(m, mb)
                mi = jnp.where(takeb, ib, mi)
            oi[pl.ds((g * GV + k) * L, L)] = mi
        return carry

    lax.fori_loop(0, QPW // (GV * L), per_group, 0)
    pltpu.sync_copy(oi, out_hbm.at[pl.ds(wid * QPW, QPW)])


_sc_nn = functools.partial(
    pl.kernel,
    out_type=jax.ShapeDtypeStruct((B * QS,), jnp.int32),
    mesh=plsc.VectorSubcoreMesh(core_axis_name="c", subcore_axis_name="s"),
    scratch_types=[
        pltpu.VMEM((QPW,), jnp.float32),
        pltpu.VMEM((QPW,), jnp.float32),
        pltpu.VMEM((QPW,), jnp.float32),
        pltpu.VMEM((M,), jnp.float32),
        pltpu.VMEM((M,), jnp.float32),
        pltpu.VMEM((M,), jnp.float32),
        pltpu.VMEM((QPW,), jnp.int32),
    ],
)(_sc_body)


CH = 1024
KT = 6
QG = KT * 1024


def _tc_body(qx_ref, qy_ref, qz_ref, rx_ref, ry_ref, rz_ref, out_ref, m_ref, mi_ref):
    r = pl.program_id(2)

    @pl.when(r == 0)
    def _():
        m_ref[...] = jnp.full((KT * 8, 128), jnp.inf, jnp.float32)
        mi_ref[...] = jnp.zeros((KT * 8, 128), jnp.int32)

    qxt = qx_ref[0, 0]
    qyt = qy_ref[0, 0]
    qzt = qz_ref[0, 0]
    base = r * CH

    def step(j, mc):
        m, mi = mc
        rx = rx_ref[0, 0, j]
        ry = ry_ref[0, 0, j]
        rz = rz_ref[0, 0, j]
        dx = qxt - rx
        dy = qyt - ry
        dz = qzt - rz
        d = dx * dx + dy * dy + dz * dz
        p = d < m
        m = jnp.minimum(m, d)
        mi = jnp.where(p, jnp.full((KT * 8, 128), base + j, jnp.int32), mi)
        return m, mi

    m, mi = lax.fori_loop(0, CH, step, (m_ref[...], mi_ref[...]), unroll=32)
    m_ref[...] = m
    mi_ref[...] = mi

    @pl.when(r == M // CH - 1)
    def _():
        out_ref[0] = mi


def _tc_nn(q, r, nb):
    qt = q.reshape(nb, 3, N // 128, 128)
    nrc = M // CH
    rt = r.reshape(nb * 3 * nrc, 1, CH)
    grid = (nb, TCQ // QG, nrc)

    def rmap(c):
        return lambda b, i, j: ((b * 3 + c) * nrc + j, 0, 0)

    out = pl.pallas_call(
        _tc_body,
        grid=grid,
        in_specs=[
            pl.BlockSpec((1, 1, QG // 128, 128), lambda b, i, j: (b, 0, i, 0)),
            pl.BlockSpec((1, 1, QG // 128, 128), lambda b, i, j: (b, 1, i, 0)),
            pl.BlockSpec((1, 1, QG // 128, 128), lambda b, i, j: (b, 2, i, 0)),
            pl.BlockSpec((1, 1, CH), rmap(0), memory_space=pltpu.SMEM),
            pl.BlockSpec((1, 1, CH), rmap(1), memory_space=pltpu.SMEM),
            pl.BlockSpec((1, 1, CH), rmap(2), memory_space=pltpu.SMEM),
        ],
        out_specs=pl.BlockSpec((1, QG // 128, 128), lambda b, i, j: (b, i, 0)),
        out_shape=jax.ShapeDtypeStruct((nb, TCQ // 128, 128), jnp.int32),
        scratch_shapes=[
            pltpu.VMEM((KT * 8, 128), jnp.float32),
            pltpu.VMEM((KT * 8, 128), jnp.int32),
        ],
        compiler_params=pltpu.CompilerParams(
            dimension_semantics=("arbitrary", "arbitrary", "arbitrary"),
        ),
    )(qt, qt, qt, rt, rt, rt)
    return out.reshape(nb, TCQ)


def kernel(S1, S2):
    q = S1.transpose(0, 2, 1)
    r = S2.transpose(0, 2, 1)
    idx_sc = _sc_nn(q.reshape(-1), r.reshape(-1)).reshape(B, QS)
    idx_tc = _tc_nn(q, r, B)
    idx = jnp.concatenate([idx_tc, idx_sc], axis=1)
    return idx.astype(jnp.int64)


# speedup vs baseline: 1.2160x; 1.0130x over previous
import functools

import jax
import jax.numpy as jnp
from jax import lax
from jax.experimental import pallas as pl
from jax.experimental.pallas import tpu as pltpu
from jax.experimental.pallas import tpu_sc as plsc

B = 4
N = 8192
M = 8192

QS = 2048
TCQ = N - QS


NW = 32
WPB = NW // B
QPW = QS // WPB
L = 16


def _sc_body(q_hbm, r_hbm, out_hbm, qx, qy, qz, rx, ry, rz, oi):
    c = lax.axis_index("c")
    s = lax.axis_index("s")
    wid = s * 2 + c
    b = wid // WPB
    qbase = TCQ + (wid % WPB) * QPW

    pltpu.sync_copy(q_hbm.at[pl.ds(b * 3 * N + 0 * N + qbase, QPW)], qx)
    pltpu.sync_copy(q_hbm.at[pl.ds(b * 3 * N + 1 * N + qbase, QPW)], qy)
    pltpu.sync_copy(q_hbm.at[pl.ds(b * 3 * N + 2 * N + qbase, QPW)], qz)
    pltpu.sync_copy(r_hbm.at[pl.ds(b * 3 * M + 0 * M, M)], rx)
    pltpu.sync_copy(r_hbm.at[pl.ds(b * 3 * M + 1 * M, M)], ry)
    pltpu.sync_copy(r_hbm.at[pl.ds(b * 3 * M + 2 * M, M)], rz)

    NA = 1
    GV = 8

    def per_group(g, carry):
        qv = [
            (
                qx[pl.ds((g * GV + k) * L, L)],
                qy[pl.ds((g * GV + k) * L, L)],
                qz[pl.ds((g * GV + k) * L, L)],
            )
            for k in range(GV)
        ]

        def per_refvec(j, mc):
            acc = [list(a) for a in zip(mc[0::2], mc[1::2])]
            rxv = rx[pl.ds(j * L, L)]
            ryv = ry[pl.ds(j * L, L)]
            rzv = rz[pl.ds(j * L, L)]
            base = jnp.full((L,), j * L, jnp.int32)
            for t in range(L):
                tv = jnp.full((L,), t, jnp.int32)
                bx = jnp.take(rxv, tv)
                by = jnp.take(ryv, tv)
                bz = jnp.take(rzv, tv)
                for k in range(GV):
                    dx = qv[k][0] - bx
                    dy = qv[k][1] - by
                    dz = qv[k][2] - bz
                    d = dx * dx + dy * dy + dz * dz
                    a = k * NA + t % NA
                    m, mi = acc[a]
                    p = d < m
                    acc[a][0] = jnp.minimum(m, d)
                    acc[a][1] = jnp.where(p, base + t, mi)
            return tuple(x for a in acc for x in a)

        init = []
        for _ in range(GV * NA):
            init.append(jnp.full((L,), jnp.inf, jnp.float32))
            init.append(jnp.zeros((L,), jnp.int32))
        accs = lax.fori_loop(0, M // L, per_refvec, tuple(init))

        for k in range(GV):
            m, mi = accs[2 * k * NA], accs[2 * k * NA + 1]
            for a in range(1, NA):
                mb = accs[2 * (k * NA + a)]
                ib = accs[2 * (k * NA + a) + 1]
                takeb = (mb < m) | ((mb == m) & (ib < mi))
                m = jnp.minimum(m, mb)
                mi = jnp.where(takeb, ib, mi)
            oi[pl.ds((g * GV + k) * L, L)] = mi
        return carry

    lax.fori_loop(0, QPW // (GV * L), per_group, 0)
    pltpu.sync_copy(oi, out_hbm.at[pl.ds(wid * QPW, QPW)])


_sc_nn = functools.partial(
    pl.kernel,
    out_type=jax.ShapeDtypeStruct((B * QS,), jnp.int32),
    mesh=plsc.VectorSubcoreMesh(core_axis_name="c", subcore_axis_name="s"),
    scratch_types=[
        pltpu.VMEM((QPW,), jnp.float32),
        pltpu.VMEM((QPW,), jnp.float32),
        pltpu.VMEM((QPW,), jnp.float32),
        pltpu.VMEM((M,), jnp.float32),
        pltpu.VMEM((M,), jnp.float32),
        pltpu.VMEM((M,), jnp.float32),
        pltpu.VMEM((QPW,), jnp.int32),
    ],
)(_sc_body)


CH = 1024
KT = 6
QG = KT * 1024


def _tc_body(qx_ref, qy_ref, qz_ref, rx_ref, ry_ref, rz_ref, out_ref, m_ref, mi_ref):
    r = pl.program_id(2)

    @pl.when(r == 0)
    def _():
        m_ref[...] = jnp.full((KT * 8, 128), jnp.inf, jnp.float32)
        mi_ref[...] = jnp.zeros((KT * 8, 128), jnp.int32)

    qxt = qx_ref[0, 0]
    qyt = qy_ref[0, 0]
    qzt = qz_ref[0, 0]
    base = r * CH

    def step(j, mc):
        m, mi = mc
        rx = rx_ref[0, 0, j]
        ry = ry_ref[0, 0, j]
        rz = rz_ref[0, 0, j]
        dx = qxt - rx
        dy = qyt - ry
        dz = qzt - rz
        d = dx * dx + dy * dy + dz * dz
        p = d < m
        m = jnp.minimum(m, d)
        mi = jnp.where(p, jnp.full((KT * 8, 128), base + j, jnp.int32), mi)
        return m, mi

    m, mi = lax.fori_loop(0, CH, step, (m_ref[...], mi_ref[...]), unroll=64)
    m_ref[...] = m
    mi_ref[...] = mi

    @pl.when(r == M // CH - 1)
    def _():
        out_ref[0] = mi


def _tc_nn(q, r, nb):
    qt = q.reshape(nb, 3, N // 128, 128)
    nrc = M // CH
    rt = r.reshape(nb * 3 * nrc, 1, CH)
    grid = (nb, TCQ // QG, nrc)

    def rmap(c):
        return lambda b, i, j: ((b * 3 + c) * nrc + j, 0, 0)

    out = pl.pallas_call(
        _tc_body,
        grid=grid,
        in_specs=[
            pl.BlockSpec((1, 1, QG // 128, 128), lambda b, i, j: (b, 0, i, 0)),
            pl.BlockSpec((1, 1, QG // 128, 128), lambda b, i, j: (b, 1, i, 0)),
            pl.BlockSpec((1, 1, QG // 128, 128), lambda b, i, j: (b, 2, i, 0)),
            pl.BlockSpec((1, 1, CH), rmap(0), memory_space=pltpu.SMEM),
            pl.BlockSpec((1, 1, CH), rmap(1), memory_space=pltpu.SMEM),
            pl.BlockSpec((1, 1, CH), rmap(2), memory_space=pltpu.SMEM),
        ],
        out_specs=pl.BlockSpec((1, QG // 128, 128), lambda b, i, j: (b, i, 0)),
        out_shape=jax.ShapeDtypeStruct((nb, TCQ // 128, 128), jnp.int32),
        scratch_shapes=[
            pltpu.VMEM((KT * 8, 128), jnp.float32),
            pltpu.VMEM((KT * 8, 128), jnp.int32),
        ],
        compiler_params=pltpu.CompilerParams(
            dimension_semantics=("arbitrary", "arbitrary", "arbitrary"),
        ),
    )(qt, qt, qt, rt, rt, rt)
    return out.reshape(nb, TCQ)


def kernel(S1, S2):
    q = S1.transpose(0, 2, 1)
    r = S2.transpose(0, 2, 1)
    idx_sc = _sc_nn(q.reshape(-1), r.reshape(-1)).reshape(B, QS)
    idx_tc = _tc_nn(q, r, B)
    idx = jnp.concatenate([idx_tc, idx_sc], axis=1)
    return idx.astype(jnp.int64)


# speedup vs baseline: 1.2228x; 1.0055x over previous
import functools

import jax
import jax.numpy as jnp
from jax import lax
from jax.experimental import pallas as pl
from jax.experimental.pallas import tpu as pltpu
from jax.experimental.pallas import tpu_sc as plsc

B = 4
N = 8192
M = 8192

QS = 2048
TCQ = N - QS


NW = 32
WPB = NW // B
QPW = QS // WPB
L = 16


def _sc_body(q_hbm, r_hbm, out_hbm, qx, qy, qz, rx, ry, rz, oi):
    c = lax.axis_index("c")
    s = lax.axis_index("s")
    wid = s * 2 + c
    b = wid // WPB
    qbase = TCQ + (wid % WPB) * QPW

    pltpu.sync_copy(q_hbm.at[pl.ds(b * 3 * N + 0 * N + qbase, QPW)], qx)
    pltpu.sync_copy(q_hbm.at[pl.ds(b * 3 * N + 1 * N + qbase, QPW)], qy)
    pltpu.sync_copy(q_hbm.at[pl.ds(b * 3 * N + 2 * N + qbase, QPW)], qz)
    pltpu.sync_copy(r_hbm.at[pl.ds(b * 3 * M + 0 * M, M)], rx)
    pltpu.sync_copy(r_hbm.at[pl.ds(b * 3 * M + 1 * M, M)], ry)
    pltpu.sync_copy(r_hbm.at[pl.ds(b * 3 * M + 2 * M, M)], rz)

    NA = 1
    GV = 8

    def per_group(g, carry):
        qv = [
            (
                qx[pl.ds((g * GV + k) * L, L)],
                qy[pl.ds((g * GV + k) * L, L)],
                qz[pl.ds((g * GV + k) * L, L)],
            )
            for k in range(GV)
        ]

        def per_refvec(j, mc):
            acc = [list(a) for a in zip(mc[0::2], mc[1::2])]
            rxv = rx[pl.ds(j * L, L)]
            ryv = ry[pl.ds(j * L, L)]
            rzv = rz[pl.ds(j * L, L)]
            base = jnp.full((L,), j * L, jnp.int32)
            for t in range(L):
                tv = jnp.full((L,), t, jnp.int32)
                bx = jnp.take(rxv, tv)
                by = jnp.take(ryv, tv)
                bz = jnp.take(rzv, tv)
                for k in range(GV):
                    dx = qv[k][0] - bx
                    dy = qv[k][1] - by
                    dz = qv[k][2] - bz
                    d = dx * dx + dy * dy + dz * dz
                    a = k * NA + t % NA
                    m, mi = acc[a]
                    p = d < m
                    acc[a][0] = jnp.minimum(m, d)
                    acc[a][1] = jnp.where(p, base + t, mi)
            return tuple(x for a in acc for x in a)

        init = []
        for _ in range(GV * NA):
            init.append(jnp.full((L,), jnp.inf, jnp.float32))
            init.append(jnp.zeros((L,), jnp.int32))
        accs = lax.fori_loop(0, M // L, per_refvec, tuple(init))

        for k in range(GV):
            m, mi = accs[2 * k * NA], accs[2 * k * NA + 1]
            for a in range(1, NA):
                mb = accs[2 * (k * NA + a)]
                ib = accs[2 * (k * NA + a) + 1]
                takeb = (mb < m) | ((mb == m) & (ib < mi))
                m = jnp.minimum(m, mb)
                mi = jnp.where(takeb, ib, mi)
            oi[pl.ds((g * GV + k) * L, L)] = mi
        return carry

    lax.fori_loop(0, QPW // (GV * L), per_group, 0)
    pltpu.sync_copy(oi, out_hbm.at[pl.ds(wid * QPW, QPW)])


_sc_nn = functools.partial(
    pl.kernel,
    out_type=jax.ShapeDtypeStruct((B * QS,), jnp.int32),
    mesh=plsc.VectorSubcoreMesh(core_axis_name="c", subcore_axis_name="s"),
    scratch_types=[
        pltpu.VMEM((QPW,), jnp.float32),
        pltpu.VMEM((QPW,), jnp.float32),
        pltpu.VMEM((QPW,), jnp.float32),
        pltpu.VMEM((M,), jnp.float32),
        pltpu.VMEM((M,), jnp.float32),
        pltpu.VMEM((M,), jnp.float32),
        pltpu.VMEM((QPW,), jnp.int32),
    ],
)(_sc_body)


CH = 2048
KT = 6
QG = KT * 1024


def _tc_body(qx_ref, qy_ref, qz_ref, rx_ref, ry_ref, rz_ref, out_ref, m_ref, mi_ref):
    r = pl.program_id(2)

    @pl.when(r == 0)
    def _():
        m_ref[...] = jnp.full((KT * 8, 128), jnp.inf, jnp.float32)
        mi_ref[...] = jnp.zeros((KT * 8, 128), jnp.int32)

    qxt = qx_ref[0, 0]
    qyt = qy_ref[0, 0]
    qzt = qz_ref[0, 0]
    base = r * CH

    def step(j, mc):
        m, mi = mc
        rx = rx_ref[0, 0, j]
        ry = ry_ref[0, 0, j]
        rz = rz_ref[0, 0, j]
        dx = qxt - rx
        dy = qyt - ry
        dz = qzt - rz
        d = dx * dx + dy * dy + dz * dz
        p = d < m
        m = jnp.minimum(m, d)
        mi = jnp.where(p, jnp.full((KT * 8, 128), base + j, jnp.int32), mi)
        return m, mi

    m, mi = lax.fori_loop(0, CH, step, (m_ref[...], mi_ref[...]), unroll=64)
    m_ref[...] = m
    mi_ref[...] = mi

    @pl.when(r == M // CH - 1)
    def _():
        out_ref[0] = mi


def _tc_nn(q, r, nb):
    qt = q.reshape(nb, 3, N // 128, 128)
    nrc = M // CH
    rt = r.reshape(nb * 3 * nrc, 1, CH)
    grid = (nb, TCQ // QG, nrc)

    def rmap(c):
        return lambda b, i, j: ((b * 3 + c) * nrc + j, 0, 0)

    out = pl.pallas_call(
        _tc_body,
        grid=grid,
        in_specs=[
            pl.BlockSpec((1, 1, QG // 128, 128), lambda b, i, j: (b, 0, i, 0)),
            pl.BlockSpec((1, 1, QG // 128, 128), lambda b, i, j: (b, 1, i, 0)),
            pl.BlockSpec((1, 1, QG // 128, 128), lambda b, i, j: (b, 2, i, 0)),
            pl.BlockSpec((1, 1, CH), rmap(0), memory_space=pltpu.SMEM),
            pl.BlockSpec((1, 1, CH), rmap(1), memory_space=pltpu.SMEM),
            pl.BlockSpec((1, 1, CH), rmap(2), memory_space=pltpu.SMEM),
        ],
        out_specs=pl.BlockSpec((1, QG // 128, 128), lambda b, i, j: (b, i, 0)),
        out_shape=jax.ShapeDtypeStruct((nb, TCQ // 128, 128), jnp.int32),
        scratch_shapes=[
            pltpu.VMEM((KT * 8, 128), jnp.float32),
            pltpu.VMEM((KT * 8, 128), jnp.int32),
        ],
        compiler_params=pltpu.CompilerParams(
            dimension_semantics=("arbitrary", "arbitrary", "arbitrary"),
        ),
    )(qt, qt, qt, rt, rt, rt)
    return out.reshape(nb, TCQ)


def kernel(S1, S2):
    q = S1.transpose(0, 2, 1)
    r = S2.transpose(0, 2, 1)
    idx_sc = _sc_nn(q.reshape(-1), r.reshape(-1)).reshape(B, QS)
    idx_tc = _tc_nn(q, r, B)
    idx = jnp.concatenate([idx_tc, idx_sc], axis=1)
    return idx.astype(jnp.int64)
